# baseline (speedup 1.0000x reference)
import functools

import numpy as np
import jax
import jax.numpy as jnp
from jax import lax
from jax.experimental import pallas as pl
from jax.experimental.pallas import tpu as pltpu
from jax.experimental.pallas import tpu_sc as plsc


def _spiral_index_map(n_circle=3, n_sequence=8, steps=(1, 2, 3)):
    width = 2 * n_circle + 1
    ci = cj = n_circle
    circle_coords = {}
    for k in range(1, n_circle + 1):
        coords = []
        i, j = ci - k, cj
        coords.append((i, j))
        moves = ([(0, 1)] * k + [(1, 0)] * (2 * k) + [(0, -1)] * (2 * k)
                 + [(-1, 0)] * (2 * k) + [(0, 1)] * (k - 1))
        for di, dj in moves:
            i += di
            j += dj
            coords.append((i, j))
        for q, cd in enumerate(coords):
            circle_coords[(k, q)] = cd
    seq_len = 1 + sum(steps)
    idx = np.zeros((n_sequence, seq_len), dtype=np.int32)
    for c in range(n_sequence):
        idx[c, 0] = ci * width + cj
        off = 1
        for k in range(1, n_circle + 1):
            s = steps[k - 1]
            pos = list(range(s * c, s * c + s))
            if c % 2 == 1:
                pos = pos[::-1]
            for q in pos:
                i, j = circle_coords[(k, q)]
                idx[c, off] = i * width + j
                off += 1
    return idx.reshape(-1)


_IDX56 = _spiral_index_map()

_NC, _NS = 2, 16
_NW = _NC * _NS
_CH = 128


@functools.cache
def _make_sc_gather(bc, n_pos, n_band):
    n_seq = _IDX56.shape[0]
    out_rows = bc * n_seq
    assert out_rows % (_NW * _CH) == 0
    rows_per_w = out_rows // _NW
    n_chunks = rows_per_w // _CH

    idx = (np.arange(bc, dtype=np.int64)[:, None] * n_pos
           + _IDX56[None, :].astype(np.int64))
    idx = idx.reshape(_NW, n_chunks, _CH).astype(np.int32)
    idx_const = jnp.asarray(idx)

    mesh = plsc.VectorSubcoreMesh(core_axis_name="c", subcore_axis_name="s",
                                  num_cores=_NC, num_subcores=_NS)

    @functools.partial(
        pl.kernel,
        out_type=jax.ShapeDtypeStruct((out_rows, n_band), jnp.float32),
        mesh=mesh,
        scratch_types=[
            pltpu.VMEM((n_chunks, _CH), jnp.int32),
            pltpu.VMEM((_CH, n_band), jnp.float32),
            pltpu.SemaphoreType.DMA,
        ],
        compiler_params=pltpu.CompilerParams(use_tc_tiling_on_sc=False),
    )
    def sc_gather(table, idxh, out, idx_v, rows_v, sem):
        wid = lax.axis_index("s") * _NC + lax.axis_index("c")
        pltpu.sync_copy(idxh.at[wid], idx_v)
        base = wid * rows_per_w

        @pl.loop(0, n_chunks)
        def _chunk(ch):
            pltpu.async_copy(table.at[idx_v.at[ch]], rows_v, sem).wait()
            pltpu.sync_copy(rows_v, out.at[pl.ds(base + ch * _CH, _CH)])

    def run(table):
        return sc_gather(table, idx_const)

    return run


def kernel(x):
    bs, c_int, w, w2, n_band = x.shape
    bc = bs * c_int
    table = x.reshape(bc * w * w2, n_band)
    out = _make_sc_gather(bc, w * w2, n_band)(table)
    return out.reshape(bs, c_int, 1, _IDX56.shape[0], n_band)


# speedup vs baseline: 1.0445x; 1.0445x over previous
import functools

import numpy as np
import jax
import jax.numpy as jnp
from jax import lax
from jax.experimental import pallas as pl
from jax.experimental.pallas import tpu as pltpu
from jax.experimental.pallas import tpu_sc as plsc


def _spiral_index_map(n_circle=3, n_sequence=8, steps=(1, 2, 3)):
    width = 2 * n_circle + 1
    ci = cj = n_circle
    circle_coords = {}
    for k in range(1, n_circle + 1):
        coords = []
        i, j = ci - k, cj
        coords.append((i, j))
        moves = ([(0, 1)] * k + [(1, 0)] * (2 * k) + [(0, -1)] * (2 * k)
                 + [(-1, 0)] * (2 * k) + [(0, 1)] * (k - 1))
        for di, dj in moves:
            i += di
            j += dj
            coords.append((i, j))
        for q, cd in enumerate(coords):
            circle_coords[(k, q)] = cd
    seq_len = 1 + sum(steps)
    idx = np.zeros((n_sequence, seq_len), dtype=np.int32)
    for c in range(n_sequence):
        idx[c, 0] = ci * width + cj
        off = 1
        for k in range(1, n_circle + 1):
            s = steps[k - 1]
            pos = list(range(s * c, s * c + s))
            if c % 2 == 1:
                pos = pos[::-1]
            for q in pos:
                i, j = circle_coords[(k, q)]
                idx[c, off] = i * width + j
                off += 1
    return idx.reshape(-1)


_IDX56 = _spiral_index_map()

_NC, _NS = 2, 16
_NW = _NC * _NS
_CH = 128


@functools.cache
def _make_sc_gather(bc, n_pos, n_band):
    n_seq = _IDX56.shape[0]
    out_rows = bc * n_seq
    assert out_rows % (_NW * _CH) == 0
    rows_per_w = out_rows // _NW
    n_chunks = rows_per_w // _CH

    idx = (np.arange(bc, dtype=np.int64)[:, None] * n_pos
           + _IDX56[None, :].astype(np.int64))
    idx = idx.reshape(_NW, n_chunks, _CH).astype(np.int32)
    idx_const = jnp.asarray(idx)

    mesh = plsc.VectorSubcoreMesh(core_axis_name="c", subcore_axis_name="s",
                                  num_cores=_NC, num_subcores=_NS)

    nbuf = 4
    assert n_chunks % nbuf == 0
    n_groups = n_chunks // nbuf

    @functools.partial(
        pl.kernel,
        out_type=jax.ShapeDtypeStruct((out_rows, n_band), jnp.float32),
        mesh=mesh,
        scratch_types=(
            [pltpu.VMEM((n_chunks, _CH), jnp.int32)]
            + [pltpu.VMEM((_CH, n_band), jnp.float32) for _ in range(nbuf)]
            + [pltpu.SemaphoreType.DMA for _ in range(2 * nbuf)]
        ),
        compiler_params=pltpu.CompilerParams(use_tc_tiling_on_sc=False),
    )
    def sc_gather(table, idxh, out, idx_v, *rest):
        rows = rest[:nbuf]
        gsem = rest[nbuf:2 * nbuf]
        wsem = rest[2 * nbuf:]
        wid = lax.axis_index("s") * _NC + lax.axis_index("c")
        pltpu.sync_copy(idxh.at[wid], idx_v)
        base = wid * rows_per_w

        def start_gather(ch, b):
            pltpu.async_copy(table.at[idx_v.at[ch]], rows[b], gsem[b])

        def start_write(ch, b):
            pltpu.async_copy(rows[b], out.at[pl.ds(base + ch * _CH, _CH)],
                             wsem[b])

        def wait_gather(b):
            pltpu.make_async_copy(table.at[idx_v.at[0]], rows[b],
                                  gsem[b]).wait()

        def wait_write(b):
            pltpu.make_async_copy(rows[b], out.at[pl.ds(base, _CH)],
                                  wsem[b]).wait()

        for b in range(nbuf):
            start_gather(b, b)

        @pl.loop(0, n_groups - 1)
        def _grp(g):
            c0 = g * nbuf
            for b in range(nbuf):
                wait_gather(b)
                start_write(c0 + b, b)
            for b in range(nbuf):
                wait_write(b)
                start_gather(c0 + nbuf + b, b)

        c0 = (n_groups - 1) * nbuf
        for b in range(nbuf):
            wait_gather(b)
            start_write(c0 + b, b)
        for b in range(nbuf):
            wait_write(b)

    def run(table):
        return sc_gather(table, idx_const)

    return run


def kernel(x):
    bs, c_int, w, w2, n_band = x.shape
    bc = bs * c_int
    table = x.reshape(bc * w * w2, n_band)
    out = _make_sc_gather(bc, w * w2, n_band)(table)
    return out.reshape(bs, c_int, 1, _IDX56.shape[0], n_band)


# speedup vs baseline: 1.9490x; 1.8659x over previous
import functools

import numpy as np
import jax
import jax.numpy as jnp
from jax import lax
from jax.experimental import pallas as pl
from jax.experimental.pallas import tpu as pltpu
from jax.experimental.pallas import tpu_sc as plsc


def _spiral_index_map(n_circle=3, n_sequence=8, steps=(1, 2, 3)):
    width = 2 * n_circle + 1
    ci = cj = n_circle
    circle_coords = {}
    for k in range(1, n_circle + 1):
        coords = []
        i, j = ci - k, cj
        coords.append((i, j))
        moves = ([(0, 1)] * k + [(1, 0)] * (2 * k) + [(0, -1)] * (2 * k)
                 + [(-1, 0)] * (2 * k) + [(0, 1)] * (k - 1))
        for di, dj in moves:
            i += di
            j += dj
            coords.append((i, j))
        for q, cd in enumerate(coords):
            circle_coords[(k, q)] = cd
    seq_len = 1 + sum(steps)
    idx = np.zeros((n_sequence, seq_len), dtype=np.int32)
    for c in range(n_sequence):
        idx[c, 0] = ci * width + cj
        off = 1
        for k in range(1, n_circle + 1):
            s = steps[k - 1]
            pos = list(range(s * c, s * c + s))
            if c % 2 == 1:
                pos = pos[::-1]
            for q in pos:
                i, j = circle_coords[(k, q)]
                idx[c, off] = i * width + j
                off += 1
    return idx.reshape(-1)


_IDX56 = _spiral_index_map()

_NC, _NS = 2, 16
_NW = _NC * _NS
_NB = 2
_VL = 16


@functools.cache
def _make_sc_scan(bc, w, n_band):
    n_seq = _IDX56.shape[0]
    assert bc % (_NW * _NB) == 0
    bc_per_w = bc // _NW
    n_groups = bc_per_w // _NB
    assert n_groups % 2 == 0

    offs = [k * _VL for k in range(n_band // _VL)]
    if n_band % _VL:
        offs.append(n_band - _VL)

    mesh = plsc.VectorSubcoreMesh(core_axis_name="c", subcore_axis_name="s",
                                  num_cores=_NC, num_subcores=_NS)

    @functools.partial(
        pl.kernel,
        out_type=jax.ShapeDtypeStruct((bc, n_seq, n_band), jnp.float32),
        mesh=mesh,
        scratch_types=(
            [pltpu.VMEM((_NB, w, w, n_band), jnp.float32) for _ in range(2)]
            + [pltpu.VMEM((_NB, n_seq, n_band), jnp.float32) for _ in range(2)]
            + [pltpu.SemaphoreType.DMA for _ in range(4)]
        ),
        compiler_params=pltpu.CompilerParams(use_tc_tiling_on_sc=True),
    )
    def sc_scan(x4, out, ib0, ib1, ob0, ob1, is0, is1, os0, os1):
        ibuf, obuf = (ib0, ib1), (ob0, ob1)
        isem, osem = (is0, is1), (os0, os1)
        wid = lax.axis_index("s") * _NC + lax.axis_index("c")
        base = wid * bc_per_w

        def start_in(g, k):
            pltpu.async_copy(x4.at[pl.ds(base + g * _NB, _NB)], ibuf[k],
                             isem[k])

        def wait_in(k):
            pltpu.make_async_copy(x4.at[pl.ds(base, _NB)], ibuf[k],
                                  isem[k]).wait()

        def start_out(g, k):
            pltpu.async_copy(obuf[k], out.at[pl.ds(base + g * _NB, _NB)],
                             osem[k])

        def wait_out(k):
            pltpu.make_async_copy(obuf[k], out.at[pl.ds(base, _NB)],
                                  osem[k]).wait()

        start_in(0, 0)
        start_in(1, 1)

        @pl.loop(0, n_groups, step=2)
        def _grp(g0):
            for k in range(2):
                g = g0 + k
                wait_in(k)

                @pl.when(g >= 2)
                def _():
                    wait_out(k)

                for b in range(_NB):
                    for s in range(n_seq):
                        p = int(_IDX56[s])
                        for o in offs:
                            obuf[k][b, s, pl.ds(o, _VL)] = (
                                ibuf[k][b, p // w, p % w, pl.ds(o, _VL)])
                start_out(g, k)

                @pl.when(g + 2 < n_groups)
                def _():
                    start_in(g + 2, k)

        wait_out(0)
        wait_out(1)

    return sc_scan


def kernel(x):
    bs, c_int, w, w2, n_band = x.shape
    bc = bs * c_int
    x4 = x.reshape(bc, w, w2, n_band)
    out = _make_sc_scan(bc, w, n_band)(x4)
    return out.reshape(bs, c_int, 1, _IDX56.shape[0], n_band)
